# baseline (speedup 1.0000x reference)
import functools

import jax
import jax.numpy as jnp
from jax import lax
from jax.experimental import pallas as pl
from jax.experimental.pallas import tpu as pltpu
from jax.experimental.pallas import tpu_sc as plsc

D = 32
NC = 2
NS = 16
NW = NC * NS


def _make_gather(nh, nb):
    blk = nb // NW
    mesh = plsc.VectorSubcoreMesh(core_axis_name="c", subcore_axis_name="s")

    @functools.partial(
        pl.kernel,
        mesh=mesh,
        out_type=jax.ShapeDtypeStruct((nh, nb, D), jnp.float32),
        scratch_types=[
            pltpu.VMEM((blk,), jnp.int32),
            pltpu.VMEM((blk,), jnp.int32),
            pltpu.VMEM((blk, D), jnp.float32),
            pltpu.VMEM((blk, D), jnp.float32),
            pltpu.SemaphoreType.DMA,
            pltpu.SemaphoreType.DMA,
            pltpu.SemaphoreType.DMA,
        ],
        compiler_params=pltpu.CompilerParams(use_tc_tiling_on_sc=False),
    )
    def gather_kernel(idx_hbm, table_hbm, out_hbm,
                      idx_v0, idx_v1, rows_v0, rows_v1, gsem, ssem0, ssem1):
        wid = lax.axis_index("s") * NC + lax.axis_index("c")
        b0 = wid * blk
        idx_vs = (idx_v0, idx_v1)
        rows_vs = (rows_v0, rows_v1)
        ssems = (ssem0, ssem1)
        gathers = [None, None]
        stores = [None, None]
        pltpu.sync_copy(idx_hbm.at[0, pl.ds(b0, blk)], idx_v0)
        gathers[0] = pltpu.async_copy(table_hbm.at[idx_v0], rows_v0, gsem)
        if nh > 1:
            pltpu.sync_copy(idx_hbm.at[1, pl.ds(b0, blk)], idx_v1)
        for h in range(nh):
            p = h % 2
            gathers[p].wait()
            stores[p] = pltpu.async_copy(
                rows_vs[p], out_hbm.at[h, pl.ds(b0, blk)], ssems[p])
            if h + 1 < nh:
                np_ = 1 - p
                if stores[np_] is not None:
                    stores[np_].wait()
                gathers[np_] = pltpu.async_copy(
                    table_hbm.at[idx_vs[np_]], rows_vs[np_], gsem)
                if h + 2 < nh:
                    pltpu.sync_copy(
                        idx_hbm.at[h + 2, pl.ds(b0, blk)], idx_vs[p])
        if nh > 1:
            stores[(nh - 2) % 2].wait()
        stores[(nh - 1) % 2].wait()

    return gather_kernel


def kernel(indices, weight):
    nb, nh = indices.shape
    idx_t = indices.T.astype(jnp.int32)
    out = _make_gather(nh, nb)(idx_t, weight)
    return out.transpose(1, 0, 2)


# speedup vs baseline: 1.0025x; 1.0025x over previous
import functools

import jax
import jax.numpy as jnp
from jax import lax
from jax.experimental import pallas as pl
from jax.experimental.pallas import tpu as pltpu
from jax.experimental.pallas import tpu_sc as plsc

D = 32
NC = 2
NS = 16
NW = NC * NS


def _make_gather(nh, nb):
    blk = nb // NW
    mesh = plsc.VectorSubcoreMesh(core_axis_name="c", subcore_axis_name="s")

    @functools.partial(
        pl.kernel,
        mesh=mesh,
        out_type=jax.ShapeDtypeStruct((nh, nb, D), jnp.float32),
        scratch_types=[
            pltpu.VMEM((blk,), jnp.int32),
            pltpu.VMEM((blk,), jnp.int32),
            pltpu.VMEM((blk, D), jnp.float32),
            pltpu.VMEM((blk, D), jnp.float32),
            pltpu.SemaphoreType.DMA,
            pltpu.SemaphoreType.DMA,
            pltpu.SemaphoreType.DMA,
        ],
        compiler_params=pltpu.CompilerParams(use_tc_tiling_on_sc=False),
    )
    def gather_kernel(idx_hbm, table_hbm, out_hbm,
                      idx_v0, idx_v1, rows_v0, rows_v1, gsem, ssem0, ssem1):
        wid = lax.axis_index("s") * NC + lax.axis_index("c")
        b0 = wid * blk
        idx_vs = (idx_v0, idx_v1)
        rows_vs = (rows_v0, rows_v1)
        ssems = (ssem0, ssem1)
        gathers = [None, None]
        stores = [None, None]
        pltpu.sync_copy(idx_hbm.at[0, pl.ds(b0, blk)], idx_v0)
        gathers[0] = pltpu.async_copy(table_hbm.at[idx_v0], rows_v0, gsem)
        if nh > 1:
            pltpu.sync_copy(idx_hbm.at[1, pl.ds(b0, blk)], idx_v1)
        for h in range(nh):
            p = h % 2
            gathers[p].wait()
            stores[p] = pltpu.async_copy(
                rows_vs[p], out_hbm.at[h, pl.ds(b0, blk)], ssems[p])
            if h + 1 < nh:
                np_ = 1 - p
                if stores[np_] is not None:
                    stores[np_].wait()
                gathers[np_] = pltpu.async_copy(
                    table_hbm.at[idx_vs[np_]], rows_vs[np_], gsem)
                if h + 2 < nh:
                    pltpu.sync_copy(
                        idx_hbm.at[h + 2, pl.ds(b0, blk)], idx_vs[p])
        if nh > 1:
            stores[(nh - 2) % 2].wait()
        stores[(nh - 1) % 2].wait()

    return gather_kernel


def kernel(indices, weight):
    nb, nh = indices.shape
    idx_t = indices.T.astype(jnp.int32)
    idx_t = jax.lax.optimization_barrier(idx_t)
    out = _make_gather(nh, nb)(idx_t, weight)
    out = jax.lax.optimization_barrier(out)
    return out.transpose(1, 0, 2)
